# baseline (speedup 1.0000x reference)
import functools

import jax
import jax.numpy as jnp
import numpy as np
from jax import lax
from jax.experimental import pallas as pl
from jax.experimental.pallas import tpu as pltpu
from jax.experimental.pallas import tpu_sc as plsc

N_ROWS = 16384
DIM = 256
N_CODES = 8192
DECAY = 0.99
EPS = 1e-5

ROW_BLK = 1024
CODE_CHUNK = 1024
N_ROW_BLKS = N_ROWS // ROW_BLK
N_CODE_CHUNKS = N_CODES // CODE_CHUNK

SC_CORES = 2
SC_SUBCORES = 16
COLS_PER_CORE = DIM // SC_CORES
ROWS_PER_SUB = N_ROWS // SC_SUBCORES
SC_CHUNK = 512
HALF_CODES = N_CODES // 2
HALF_PAD = HALF_CODES + 8
HALF_PER_SUB = HALF_CODES // SC_SUBCORES


def _stats_body(x_ref, emb_ref, mean_ref, var_ref, e2_ref):
    i = pl.program_id(0)
    blk = x_ref[...]
    s = jnp.sum(blk, axis=0, keepdims=True)
    sq = jnp.sum(blk * blk, axis=0, keepdims=True)
    eb = emb_ref[...]
    e2_ref[...] = jnp.sum(eb * eb, axis=1, keepdims=True)

    @pl.when(i == 0)
    def _():
        mean_ref[...] = s
        var_ref[...] = sq

    @pl.when(i > 0)
    def _():
        mean_ref[...] += s
        var_ref[...] += sq

    @pl.when(i == N_ROW_BLKS - 1)
    def _():
        m = mean_ref[...] * (1.0 / N_ROWS)
        v = var_ref[...] * (1.0 / N_ROWS) - m * m
        mean_ref[...] = m
        var_ref[...] = v


def _stats(x, emb):
    eblk = N_CODES // N_ROW_BLKS
    return pl.pallas_call(
        _stats_body,
        grid=(N_ROW_BLKS,),
        in_specs=[pl.BlockSpec((ROW_BLK, DIM), lambda i: (i, 0)),
                  pl.BlockSpec((eblk, DIM), lambda i: (i, 0))],
        out_specs=[
            pl.BlockSpec((1, DIM), lambda i: (0, 0)),
            pl.BlockSpec((1, DIM), lambda i: (0, 0)),
            pl.BlockSpec((eblk, 1), lambda i: (i, 0)),
        ],
        out_shape=[
            jax.ShapeDtypeStruct((1, DIM), jnp.float32),
            jax.ShapeDtypeStruct((1, DIM), jnp.float32),
            jax.ShapeDtypeStruct((N_CODES, 1), jnp.float32),
        ],
    )(x, emb)


def _assign_body(x_ref, emb_ref, mean_ref, var_ref, e2_ref, xn_ref, idxlo_ref,
                 idxhi_ref, iota_ref, xm2_ref):
    i = pl.program_id(0)

    @pl.when(i == 0)
    def _():
        iota_ref[...] = lax.broadcasted_iota(
            jnp.int32, (1, N_CODES), 1).astype(jnp.float32)

    rstd = lax.rsqrt(var_ref[...] + EPS)
    xn = (x_ref[...] - mean_ref[...]) * rstd
    xn_ref[...] = xn
    xm2_ref[...] = (xn * (-2.0)).astype(jnp.bfloat16)
    xm2 = xm2_ref[...]

    runmin = jnp.full((ROW_BLK, 1), jnp.inf, dtype=jnp.float32)
    runidx = jnp.zeros((ROW_BLK, 1), dtype=jnp.float32)
    for c in range(N_CODE_CHUNKS):
        lo = c * CODE_CHUNK
        d = jnp.dot(xm2, emb_ref[:, lo:lo + CODE_CHUNK],
                    preferred_element_type=jnp.float32)
        d = d + e2_ref[:, lo:lo + CODE_CHUNK]
        m = jnp.min(d, axis=1, keepdims=True)
        cand = jnp.min(jnp.where(d == m, iota_ref[:, lo:lo + CODE_CHUNK],
                                 jnp.float32(1e9)),
                       axis=1, keepdims=True)
        upd = m < runmin
        runidx = jnp.where(upd, cand, runidx)
        runmin = jnp.where(upd, m, runmin)
    runidx = runidx.astype(jnp.int32)
    lo_ok = runidx < HALF_CODES
    idxlo_ref[...] = jnp.where(lo_ok, runidx, HALF_CODES)
    idxhi_ref[...] = jnp.where(lo_ok, HALF_CODES, runidx - HALF_CODES)


def _assign(x, emb_bf, mean, var, e2r):
    nrows = x.shape[0]
    return pl.pallas_call(
        _assign_body,
        grid=(nrows // ROW_BLK,),
        in_specs=[
            pl.BlockSpec((ROW_BLK, DIM), lambda i: (i, 0)),
            pl.BlockSpec((DIM, N_CODES), lambda i: (0, 0)),
            pl.BlockSpec((1, DIM), lambda i: (0, 0)),
            pl.BlockSpec((1, DIM), lambda i: (0, 0)),
            pl.BlockSpec((1, N_CODES), lambda i: (0, 0)),
        ],
        out_specs=[
            pl.BlockSpec((ROW_BLK, DIM), lambda i: (i, 0)),
            pl.BlockSpec((ROW_BLK, 1), lambda i: (i, 0)),
            pl.BlockSpec((ROW_BLK, 1), lambda i: (i, 0)),
        ],
        out_shape=[
            jax.ShapeDtypeStruct((nrows, DIM), jnp.float32),
            jax.ShapeDtypeStruct((nrows, 1), jnp.int32),
            jax.ShapeDtypeStruct((nrows, 1), jnp.int32),
        ],
        scratch_shapes=[pltpu.VMEM((1, N_CODES), jnp.float32),
                        pltpu.VMEM((ROW_BLK, DIM), jnp.bfloat16)],
    )(x, emb_bf, mean, var, e2r)


def _scatter(xn, idxlo, idxhi, zeros2d, zeros1d, ones1d):
    nrows = xn.shape[0]
    rows_per_sub = nrows // SC_SUBCORES

    def _scatter_body(xn_hbm, idxlo_hbm, idxhi_hbm, z_hbm, z1_hbm, on_hbm,
                      dw_hbm, cnt_hbm, dwsh, cntsh, idx0_v, idx1_v, rows_v,
                      ones_v, sems):
        c = lax.axis_index("core")
        s = lax.axis_index("subcore")
        col0 = c * COLS_PER_CORE
        r0 = s * HALF_PER_SUB
        base = s * rows_per_sub

        rows_d = pltpu.async_copy(
            xn_hbm.at[pl.ds(base, rows_per_sub), pl.ds(col0, COLS_PER_CORE)],
            rows_v, sems.at[0])
        idx0_d = pltpu.async_copy(idxlo_hbm.at[pl.ds(base, rows_per_sub)],
                                  idx0_v, sems.at[1])
        idx1_d = pltpu.async_copy(idxhi_hbm.at[pl.ds(base, rows_per_sub)],
                                  idx1_v, sems.at[2])
        ones_d = pltpu.async_copy(on_hbm, ones_v, sems.at[3])
        zc_d = pltpu.async_copy(z1_hbm.at[pl.ds(r0, HALF_PER_SUB)],
                                cntsh.at[pl.ds(r0, HALF_PER_SUB)], sems.at[4])

        for p, idx_v, idx_d in ((0, idx0_v, idx0_d), (1, idx1_v, idx1_d)):
            z_d = pltpu.async_copy(
                z_hbm.at[pl.ds(r0, HALF_PER_SUB), pl.ds(col0, COLS_PER_CORE)],
                dwsh.at[pl.ds(r0, HALF_PER_SUB)], sems.at[5])
            if p == 0:
                zc_d.wait()
            z_d.wait()
            plsc.subcore_barrier()

            if p == 0:
                rows_d.wait()
            idx_d.wait()
            s_d = pltpu.async_copy(rows_v, dwsh.at[idx_v], sems.at[6], add=True)

            @pl.when(c == p)
            def _():
                if p == 0:
                    ones_d.wait()
                cnt_d = pltpu.async_copy(ones_v, cntsh.at[idx_v], sems.at[7],
                                         add=True)
                cnt_d.wait()

            s_d.wait()
            plsc.subcore_barrier()

            f_d = pltpu.async_copy(
                dwsh.at[pl.ds(r0, HALF_PER_SUB)],
                dw_hbm.at[pl.ds(p * HALF_CODES + r0, HALF_PER_SUB),
                          pl.ds(col0, COLS_PER_CORE)], sems.at[8])

            @pl.when(c == p)
            def _():
                cf_d = pltpu.async_copy(
                    cntsh.at[pl.ds(r0, HALF_PER_SUB)],
                    cnt_hbm.at[pl.ds(p * HALF_CODES + r0, HALF_PER_SUB)],
                    sems.at[9])
                cf_d.wait()

            f_d.wait()
            plsc.subcore_barrier()

    mesh = plsc.VectorSubcoreMesh(core_axis_name="core", subcore_axis_name="subcore")
    f = pl.kernel(
        _scatter_body,
        out_type=[
            jax.ShapeDtypeStruct((N_CODES, DIM), jnp.float32),
            jax.ShapeDtypeStruct((N_CODES,), jnp.float32),
        ],
        mesh=mesh,
        scratch_types=[
            pltpu.VMEM_SHARED((HALF_PAD, COLS_PER_CORE), jnp.float32),
            pltpu.VMEM_SHARED((HALF_PAD,), jnp.float32),
            pltpu.VMEM((rows_per_sub,), jnp.int32),
            pltpu.VMEM((rows_per_sub,), jnp.int32),
            pltpu.VMEM((rows_per_sub, COLS_PER_CORE), jnp.float32),
            pltpu.VMEM((rows_per_sub,), jnp.float32),
            pltpu.SemaphoreType.DMA((10,)),
        ],
    )
    return f(xn, idxlo, idxhi, zeros2d, zeros1d, ones1d)


def _finalize_body(emb_ref, dwa_ref, dwb_ref, cs_ref, cnta_ref, cntb_ref,
                   mean_ref, var_ref, out_ref):
    cs = cs_ref[...]
    dw = dwa_ref[...] + dwb_ref[...]
    cnt = cnta_ref[...] + cntb_ref[...]
    num = cs * emb_ref[...] * DECAY + (1.0 - DECAY) * dw
    den = cs * DECAY + (1.0 - DECAY) * cnt
    unbiased = var_ref[...] * (N_ROWS / (N_ROWS - 1.0))
    running_std = jnp.sqrt(unbiased + EPS)
    out_ref[...] = num / den * running_std + mean_ref[...]


def _finalize(emb, dwa, dwb, cs2d, cnta, cntb, mean, var):
    nblk = 8
    blk = N_CODES // nblk
    return pl.pallas_call(
        _finalize_body,
        grid=(nblk,),
        in_specs=[
            pl.BlockSpec((blk, DIM), lambda i: (i, 0)),
            pl.BlockSpec((blk, DIM), lambda i: (i, 0)),
            pl.BlockSpec((blk, DIM), lambda i: (i, 0)),
            pl.BlockSpec((blk, 1), lambda i: (i, 0)),
            pl.BlockSpec((blk, 1), lambda i: (i, 0)),
            pl.BlockSpec((blk, 1), lambda i: (i, 0)),
            pl.BlockSpec((1, DIM), lambda i: (0, 0)),
            pl.BlockSpec((1, DIM), lambda i: (0, 0)),
        ],
        out_specs=pl.BlockSpec((blk, DIM), lambda i: (i, 0)),
        out_shape=jax.ShapeDtypeStruct((N_CODES, DIM), jnp.float32),
    )(emb, dwa, dwb, cs2d, cnta, cntb, mean, var)


_ZEROS2D = np.zeros((HALF_CODES, DIM), dtype=np.float32)
_ZEROS1D = np.zeros((HALF_CODES,), dtype=np.float32)
_ONES1D = np.ones((SC_CHUNK,), dtype=np.float32)


@jax.jit
def kernel(x, vq_embedding, vq_cluster_size):
    mean, var, e2 = _stats(x, vq_embedding)
    emb_bf = vq_embedding.T.astype(jnp.bfloat16)
    e2r = e2.reshape(1, N_CODES)
    half = N_ROWS // 2
    xn0, lo0, hi0 = _assign(x[:half], emb_bf, mean, var, e2r)
    dw0, cnt0 = _scatter(xn0, lo0.reshape(half), hi0.reshape(half),
                         _ZEROS2D, _ZEROS1D, _ONES1D)
    xn1, lo1, hi1 = _assign(x[half:], emb_bf, mean, var, e2r)
    dw1, cnt1 = _scatter(xn1, lo1.reshape(half), hi1.reshape(half),
                         _ZEROS2D, _ZEROS1D, _ONES1D)
    cs2d = vq_cluster_size.reshape(N_CODES, 1)
    return _finalize(vq_embedding, dw0, dw1, cs2d,
                     cnt0.reshape(N_CODES, 1), cnt1.reshape(N_CODES, 1),
                     mean, var)


# speedup vs baseline: 1.0255x; 1.0255x over previous
import functools

import jax
import jax.numpy as jnp
import numpy as np
from jax import lax
from jax.experimental import pallas as pl
from jax.experimental.pallas import tpu as pltpu
from jax.experimental.pallas import tpu_sc as plsc

N_ROWS = 16384
DIM = 256
N_CODES = 8192
DECAY = 0.99
EPS = 1e-5

ROW_BLK = 1024
CODE_CHUNK = 1024
N_ROW_BLKS = N_ROWS // ROW_BLK
N_CODE_CHUNKS = N_CODES // CODE_CHUNK

SC_CORES = 2
SC_SUBCORES = 16
COLS_PER_CORE = DIM // SC_CORES
ROWS_PER_SUB = N_ROWS // SC_SUBCORES
SC_CHUNK = 512
HALF_CODES = N_CODES // 2
HALF_PAD = HALF_CODES + 8
HALF_PER_SUB = HALF_CODES // SC_SUBCORES


def _stats_body(x_ref, mean_ref, var_ref):
    i = pl.program_id(0)
    blk = x_ref[...]
    s = jnp.sum(blk, axis=0, keepdims=True)
    sq = jnp.sum(blk * blk, axis=0, keepdims=True)

    @pl.when(i == 0)
    def _():
        mean_ref[...] = s
        var_ref[...] = sq

    @pl.when(i > 0)
    def _():
        mean_ref[...] += s
        var_ref[...] += sq

    @pl.when(i == N_ROW_BLKS - 1)
    def _():
        m = mean_ref[...] * (1.0 / N_ROWS)
        v = var_ref[...] * (1.0 / N_ROWS) - m * m
        mean_ref[...] = m
        var_ref[...] = v


def _stats(x):
    return pl.pallas_call(
        _stats_body,
        grid=(N_ROW_BLKS,),
        in_specs=[pl.BlockSpec((ROW_BLK, DIM), lambda i: (i, 0))],
        out_specs=[
            pl.BlockSpec((1, DIM), lambda i: (0, 0)),
            pl.BlockSpec((1, DIM), lambda i: (0, 0)),
        ],
        out_shape=[
            jax.ShapeDtypeStruct((1, DIM), jnp.float32),
            jax.ShapeDtypeStruct((1, DIM), jnp.float32),
        ],
    )(x)


def _assign_body(x_ref, emb_ref, mean_ref, var_ref, xn_ref, idxlo_ref,
                 idxhi_ref, e2_ref, iota_ref):
    i = pl.program_id(0)

    @pl.when(i == 0)
    def _():
        et = emb_ref[...].astype(jnp.float32)
        e2_ref[...] = jnp.sum(et * et, axis=0, keepdims=True)
        iota_ref[...] = lax.broadcasted_iota(
            jnp.int32, (1, N_CODES), 1).astype(jnp.float32)

    rstd = lax.rsqrt(var_ref[...] + EPS)
    xn = (x_ref[...] - mean_ref[...]) * rstd
    xn_ref[...] = xn
    xm2 = (xn * (-2.0)).astype(jnp.bfloat16)

    runmin = jnp.full((ROW_BLK, 1), jnp.inf, dtype=jnp.float32)
    runidx = jnp.zeros((ROW_BLK, 1), dtype=jnp.float32)
    for c in range(N_CODE_CHUNKS):
        lo = c * CODE_CHUNK
        d = jnp.dot(xm2, emb_ref[:, lo:lo + CODE_CHUNK],
                    preferred_element_type=jnp.float32)
        d = d + e2_ref[:, lo:lo + CODE_CHUNK]
        m = jnp.min(d, axis=1, keepdims=True)
        cand = jnp.min(jnp.where(d == m, iota_ref[:, lo:lo + CODE_CHUNK],
                                 jnp.float32(1e9)),
                       axis=1, keepdims=True)
        upd = m < runmin
        runidx = jnp.where(upd, cand, runidx)
        runmin = jnp.where(upd, m, runmin)
    runidx = runidx.astype(jnp.int32)
    lo_ok = runidx < HALF_CODES
    idxlo_ref[...] = jnp.where(lo_ok, runidx, HALF_CODES)
    idxhi_ref[...] = jnp.where(lo_ok, HALF_CODES, runidx - HALF_CODES)


def _assign(x, emb_bf, mean, var):
    nrows = x.shape[0]
    return pl.pallas_call(
        _assign_body,
        grid=(nrows // ROW_BLK,),
        in_specs=[
            pl.BlockSpec((ROW_BLK, DIM), lambda i: (i, 0)),
            pl.BlockSpec((DIM, N_CODES), lambda i: (0, 0)),
            pl.BlockSpec((1, DIM), lambda i: (0, 0)),
            pl.BlockSpec((1, DIM), lambda i: (0, 0)),
        ],
        out_specs=[
            pl.BlockSpec((ROW_BLK, DIM), lambda i: (i, 0)),
            pl.BlockSpec((ROW_BLK, 1), lambda i: (i, 0)),
            pl.BlockSpec((ROW_BLK, 1), lambda i: (i, 0)),
        ],
        out_shape=[
            jax.ShapeDtypeStruct((nrows, DIM), jnp.float32),
            jax.ShapeDtypeStruct((nrows, 1), jnp.int32),
            jax.ShapeDtypeStruct((nrows, 1), jnp.int32),
        ],
        scratch_shapes=[pltpu.VMEM((1, N_CODES), jnp.float32),
                        pltpu.VMEM((1, N_CODES), jnp.float32)],
    )(x, emb_bf, mean, var)


def _scatter(xn, idxlo, idxhi, zeros2d, zeros1d, ones1d):
    nrows = xn.shape[0]
    rows_per_sub = nrows // SC_SUBCORES

    def _scatter_body(xn_hbm, idxlo_hbm, idxhi_hbm, z_hbm, z1_hbm, on_hbm,
                      dw_hbm, cnt_hbm, dwsh, cntsh, idx0_v, idx1_v, rows_v,
                      ones_v, sems):
        c = lax.axis_index("core")
        s = lax.axis_index("subcore")
        col0 = c * COLS_PER_CORE
        r0 = s * HALF_PER_SUB
        base = s * rows_per_sub

        rows_d = pltpu.async_copy(
            xn_hbm.at[pl.ds(base, rows_per_sub), pl.ds(col0, COLS_PER_CORE)],
            rows_v, sems.at[0])
        idx0_d = pltpu.async_copy(idxlo_hbm.at[pl.ds(base, rows_per_sub)],
                                  idx0_v, sems.at[1])
        idx1_d = pltpu.async_copy(idxhi_hbm.at[pl.ds(base, rows_per_sub)],
                                  idx1_v, sems.at[2])
        ones_d = pltpu.async_copy(on_hbm, ones_v, sems.at[3])
        zc_d = pltpu.async_copy(z1_hbm.at[pl.ds(r0, HALF_PER_SUB)],
                                cntsh.at[pl.ds(r0, HALF_PER_SUB)], sems.at[4])

        for p, idx_v, idx_d in ((0, idx0_v, idx0_d), (1, idx1_v, idx1_d)):
            z_d = pltpu.async_copy(
                z_hbm.at[pl.ds(r0, HALF_PER_SUB), pl.ds(col0, COLS_PER_CORE)],
                dwsh.at[pl.ds(r0, HALF_PER_SUB)], sems.at[5])
            if p == 0:
                zc_d.wait()
            z_d.wait()
            plsc.subcore_barrier()

            if p == 0:
                rows_d.wait()
            idx_d.wait()
            s_d = pltpu.async_copy(rows_v, dwsh.at[idx_v], sems.at[6], add=True)

            @pl.when(c == p)
            def _():
                if p == 0:
                    ones_d.wait()
                cnt_d = pltpu.async_copy(ones_v, cntsh.at[idx_v], sems.at[7],
                                         add=True)
                cnt_d.wait()

            s_d.wait()
            plsc.subcore_barrier()

            f_d = pltpu.async_copy(
                dwsh.at[pl.ds(r0, HALF_PER_SUB)],
                dw_hbm.at[pl.ds(p * HALF_CODES + r0, HALF_PER_SUB),
                          pl.ds(col0, COLS_PER_CORE)], sems.at[8])

            @pl.when(c == p)
            def _():
                cf_d = pltpu.async_copy(
                    cntsh.at[pl.ds(r0, HALF_PER_SUB)],
                    cnt_hbm.at[pl.ds(p * HALF_CODES + r0, HALF_PER_SUB)],
                    sems.at[9])
                cf_d.wait()

            f_d.wait()
            plsc.subcore_barrier()

    mesh = plsc.VectorSubcoreMesh(core_axis_name="core", subcore_axis_name="subcore")
    f = pl.kernel(
        _scatter_body,
        out_type=[
            jax.ShapeDtypeStruct((N_CODES, DIM), jnp.float32),
            jax.ShapeDtypeStruct((N_CODES,), jnp.float32),
        ],
        mesh=mesh,
        scratch_types=[
            pltpu.VMEM_SHARED((HALF_PAD, COLS_PER_CORE), jnp.float32),
            pltpu.VMEM_SHARED((HALF_PAD,), jnp.float32),
            pltpu.VMEM((rows_per_sub,), jnp.int32),
            pltpu.VMEM((rows_per_sub,), jnp.int32),
            pltpu.VMEM((rows_per_sub, COLS_PER_CORE), jnp.float32),
            pltpu.VMEM((rows_per_sub,), jnp.float32),
            pltpu.SemaphoreType.DMA((10,)),
        ],
    )
    return f(xn, idxlo, idxhi, zeros2d, zeros1d, ones1d)


def _finalize_body(emb_ref, dwa_ref, dwb_ref, cs_ref, cnta_ref, cntb_ref,
                   mean_ref, var_ref, out_ref):
    cs = cs_ref[...]
    dw = dwa_ref[...] + dwb_ref[...]
    cnt = cnta_ref[...] + cntb_ref[...]
    num = cs * emb_ref[...] * DECAY + (1.0 - DECAY) * dw
    den = cs * DECAY + (1.0 - DECAY) * cnt
    unbiased = var_ref[...] * (N_ROWS / (N_ROWS - 1.0))
    running_std = jnp.sqrt(unbiased + EPS)
    out_ref[...] = num / den * running_std + mean_ref[...]


def _finalize(emb, dwa, dwb, cs2d, cnta, cntb, mean, var):
    nblk = 8
    blk = N_CODES // nblk
    return pl.pallas_call(
        _finalize_body,
        grid=(nblk,),
        in_specs=[
            pl.BlockSpec((blk, DIM), lambda i: (i, 0)),
            pl.BlockSpec((blk, DIM), lambda i: (i, 0)),
            pl.BlockSpec((blk, DIM), lambda i: (i, 0)),
            pl.BlockSpec((blk, 1), lambda i: (i, 0)),
            pl.BlockSpec((blk, 1), lambda i: (i, 0)),
            pl.BlockSpec((blk, 1), lambda i: (i, 0)),
            pl.BlockSpec((1, DIM), lambda i: (0, 0)),
            pl.BlockSpec((1, DIM), lambda i: (0, 0)),
        ],
        out_specs=pl.BlockSpec((blk, DIM), lambda i: (i, 0)),
        out_shape=jax.ShapeDtypeStruct((N_CODES, DIM), jnp.float32),
    )(emb, dwa, dwb, cs2d, cnta, cntb, mean, var)


_ZEROS2D = np.zeros((HALF_CODES, DIM), dtype=np.float32)
_ZEROS1D = np.zeros((HALF_CODES,), dtype=np.float32)
_ONES1D = np.ones((SC_CHUNK,), dtype=np.float32)


@jax.jit
def kernel(x, vq_embedding, vq_cluster_size):
    mean, var = _stats(x)
    emb_bf = vq_embedding.T.astype(jnp.bfloat16)
    half = N_ROWS // 2
    xn0, lo0, hi0 = _assign(x[:half], emb_bf, mean, var)
    dw0, cnt0 = _scatter(xn0, lo0.reshape(half), hi0.reshape(half),
                         _ZEROS2D, _ZEROS1D, _ONES1D)
    xn1, lo1, hi1 = _assign(x[half:], emb_bf, mean, var)
    dw1, cnt1 = _scatter(xn1, lo1.reshape(half), hi1.reshape(half),
                         _ZEROS2D, _ZEROS1D, _ONES1D)
    cs2d = vq_cluster_size.reshape(N_CODES, 1)
    return _finalize(vq_embedding, dw0, dw1, cs2d,
                     cnt0.reshape(N_CODES, 1), cnt1.reshape(N_CODES, 1),
                     mean, var)
